# baseline (speedup 1.0000x reference)
import functools

import jax
import jax.numpy as jnp
from jax import lax
from jax.experimental import pallas as pl
from jax.experimental.pallas import tpu as pltpu
from jax.experimental.pallas import tpu_sc as plsc

VOCAB = 50265
HID = 768
MAXPOS = 2050
TYPEV = 2
PAD = 1
EPS = 1e-12
B = 4
S = 2048
N = B * S
_NH = N // 2

_SLAB = 128
_BLK = 512
_NBLK = S // _BLK
_KSLAB = _BLK + _SLAB
_TAB_PAD = 2304

_NC = 2
_NS = 16
_NW = _NC * _NS
_B_PER_W = _NH // _NW
_CH = 128


def _posid_body(ids_ref, ptab_ref, type_ref, out_ref, ss_ref, tab_ref):
    ids = ids_ref[...]
    mask = (ids != PAD)
    maskb = mask.astype(jnp.bfloat16)
    ri = lax.broadcasted_iota(jnp.int32, (S, S), 0)
    ci = lax.broadcasted_iota(jnp.int32, (S, S), 1)
    tri = (ri <= ci).astype(jnp.bfloat16)
    inc = lax.dot_general(maskb, tri, (((1,), (0,)), ((), ())),
                          preferred_element_type=jnp.float32)
    inc = inc.astype(jnp.int32)
    out_ref[...] = inc * mask.astype(jnp.int32) + PAD
    ss_cols = [jnp.zeros((B, 1), jnp.int32)]
    for k in range(1, _NBLK):
        ss_cols.append((inc[:, k * _BLK - 1:k * _BLK] + 1) // _SLAB)
    ss_ref[...] = jnp.concatenate(ss_cols, axis=1)
    t = ptab_ref[...] + type_ref[0, :][None, :]
    tab_ref[pl.ds(0, 2048), :] = t[0:2048, :].astype(jnp.bfloat16)
    tail = jnp.concatenate(
        [t[2048:MAXPOS, :], jnp.zeros((16 - (MAXPOS - 2048), HID),
                                      jnp.float32)], axis=0)
    tab_ref[pl.ds(2048, 16), :] = tail.astype(jnp.bfloat16)
    tab_ref[pl.ds(2064, _TAB_PAD - 2064), :] = jnp.zeros(
        (_TAB_PAD - 2064, HID), jnp.bfloat16)


_posid_call = pl.pallas_call(
    _posid_body,
    out_shape=(jax.ShapeDtypeStruct((B, S), jnp.int32),
               jax.ShapeDtypeStruct((B, _NBLK), jnp.int32),
               jax.ShapeDtypeStruct((_TAB_PAD, HID), jnp.bfloat16)),
)


def _gather_body(word_hbm, wid_hbm, ow_hbm, idx_v, rows_v, sem):
    w = lax.axis_index("s") * _NC + lax.axis_index("c")
    base = w * _B_PER_W

    @pl.loop(0, _B_PER_W, step=_CH)
    def _(c):
        off = base + c
        pltpu.sync_copy(wid_hbm.at[pl.ds(off, _CH)], idx_v)
        pltpu.async_copy(word_hbm.at[idx_v], rows_v, sem).wait()
        pltpu.sync_copy(rows_v, ow_hbm.at[pl.ds(off, _CH)])


@functools.cache
def _gather_call():
    return functools.partial(
        pl.kernel,
        out_type=jax.ShapeDtypeStruct((_NH, HID), jnp.float32),
        mesh=plsc.VectorSubcoreMesh(core_axis_name="c", subcore_axis_name="s"),
        scratch_types=[
            pltpu.VMEM((_CH,), jnp.int32),
            pltpu.VMEM((_CH, HID), jnp.float32),
            pltpu.SemaphoreType.DMA,
        ],
    )(_gather_body)


def _make_ln_body(boff, aliased):
    def _ln_body(*refs):
        if aliased:
            (ss_ref, pid_ref, gw_ref, tab_ref, g_ref, b_ref,
             _prev_ref, o_ref) = refs
        else:
            (ss_ref, pid_ref, gw_ref, tab_ref, g_ref, b_ref,
             o_ref) = refs
        k = pl.program_id(0)
        b = pl.program_id(1) + boff
        ssv = ss_ref[b, k]
        pidr = pid_ref[0, 0, :][None, :]

        slab = tab_ref[pl.ds(ssv * _SLAB, _KSLAB), :]
        lpid = pidr - ssv * _SLAB
        rows = lax.broadcasted_iota(jnp.int32, (_KSLAB, _BLK), 0)
        onehot_t = (lpid == rows).astype(jnp.bfloat16)
        acc = lax.dot_general(onehot_t, slab, (((0,), (0,)), ((), ())),
                              preferred_element_type=jnp.float32)
        sel = lax.dot_general(onehot_t, jnp.ones((_KSLAB, 1), jnp.bfloat16),
                              (((0,), (0,)), ((), ())),
                              preferred_element_type=jnp.float32)
        padrow = (1.0 - sel) * tab_ref[PAD, :][None, :].astype(jnp.float32)
        x = gw_ref[...] + acc + padrow
        mean = jnp.mean(x, axis=-1, keepdims=True)
        xc = x - mean
        var = jnp.mean(xc * xc, axis=-1, keepdims=True)
        o_ref[...] = (xc * lax.rsqrt(var + EPS)) * g_ref[0, :][None, :] \
            + b_ref[0, :][None, :]
    return _ln_body


def _make_ln_call(boff, aliased):
    def _ob(k, b, ss):
        return (_NBLK * (b + boff) + k, 0)

    def _pb(k, b, ss):
        return (_NBLK * (b + boff) + k, 0, 0)

    in_specs = [
        pl.BlockSpec((1, 1, _BLK), _pb),
        pl.BlockSpec((_BLK, HID), lambda k, b, ss: (_NBLK * b + k, 0)),
        pl.BlockSpec((_TAB_PAD, HID), lambda k, b, ss: (0, 0)),
        pl.BlockSpec((1, HID), lambda k, b, ss: (0, 0)),
        pl.BlockSpec((1, HID), lambda k, b, ss: (0, 0)),
    ]
    kwargs = {}
    if aliased:
        in_specs.append(pl.BlockSpec(memory_space=pl.ANY))
        kwargs["input_output_aliases"] = {6: 0}
    return pl.pallas_call(
        _make_ln_body(boff, aliased),
        grid_spec=pltpu.PrefetchScalarGridSpec(
            num_scalar_prefetch=1,
            grid=(_NBLK, B // 2),
            in_specs=in_specs,
            out_specs=pl.BlockSpec((_BLK, HID), _ob),
        ),
        out_shape=jax.ShapeDtypeStruct((N, HID), jnp.float32),
        **kwargs,
    )


_ln_call_a = _make_ln_call(0, aliased=False)
_ln_call_b = _make_ln_call(B // 2, aliased=True)


def kernel(input_ids, word_emb, pos_emb, type_emb, ln_gamma, ln_beta):
    position_ids, ss, tab = _posid_call(input_ids, pos_emb, type_emb)
    ids_flat = input_ids.reshape(N)
    pid_col = position_ids.reshape(N // _BLK, 1, _BLK)
    gw_a = _gather_call()(word_emb, ids_flat[:_NH])
    gw_b = _gather_call()(word_emb, ids_flat[_NH:])
    g2 = ln_gamma.reshape(1, HID)
    b2 = ln_beta.reshape(1, HID)
    out_a = _ln_call_a(ss, pid_col, gw_a, tab, g2, b2)
    out = _ln_call_b(ss, pid_col, gw_b, tab, g2, b2, out_a)
    return out.reshape(B, S, HID), position_ids


# speedup vs baseline: 1.0202x; 1.0202x over previous
import functools

import jax
import jax.numpy as jnp
from jax import lax
from jax.experimental import pallas as pl
from jax.experimental.pallas import tpu as pltpu
from jax.experimental.pallas import tpu_sc as plsc

VOCAB = 50265
HID = 768
MAXPOS = 2050
TYPEV = 2
PAD = 1
EPS = 1e-12
B = 4
S = 2048
N = B * S
_NH = N // 2

_SLAB = 256
_BLK = 512
_NBLK = S // _BLK
_KSLAB = _BLK + _SLAB
_TAB_PAD = 2304

_NC = 2
_NS = 16
_NW = _NC * _NS
_B_PER_W = _NH // _NW
_CH = 128


def _posid_body(ids_ref, ptab_ref, out_ref, ss_ref, tab_ref):
    ids = ids_ref[...]
    mask = (ids != PAD)
    maskb = mask.astype(jnp.bfloat16)
    ri = lax.broadcasted_iota(jnp.int32, (S, S), 0)
    ci = lax.broadcasted_iota(jnp.int32, (S, S), 1)
    tri = (ri <= ci).astype(jnp.bfloat16)
    inc = lax.dot_general(maskb, tri, (((1,), (0,)), ((), ())),
                          preferred_element_type=jnp.float32)
    inc = inc.astype(jnp.int32)
    out_ref[...] = inc * mask.astype(jnp.int32) + PAD
    ss_cols = [jnp.zeros((B, 1), jnp.int32)]
    for k in range(1, _NBLK):
        ss_cols.append((inc[:, k * _BLK - 1:k * _BLK] + 1) // _SLAB)
    ss_ref[...] = jnp.concatenate(ss_cols, axis=1)
    t = ptab_ref[...]
    tab_ref[pl.ds(0, 2048), :] = t[0:2048, :].astype(jnp.bfloat16)
    tail = jnp.concatenate(
        [t[2048:MAXPOS, :], jnp.zeros((16 - (MAXPOS - 2048), HID),
                                      jnp.float32)], axis=0)
    tab_ref[pl.ds(2048, 16), :] = tail.astype(jnp.bfloat16)
    tab_ref[pl.ds(2064, _TAB_PAD - 2064), :] = jnp.zeros(
        (_TAB_PAD - 2064, HID), jnp.bfloat16)


_posid_call = pl.pallas_call(
    _posid_body,
    out_shape=(jax.ShapeDtypeStruct((B, S), jnp.int32),
               jax.ShapeDtypeStruct((B, _NBLK), jnp.int32),
               jax.ShapeDtypeStruct((_TAB_PAD, HID), jnp.bfloat16)),
)


def _gather_body(word_hbm, wid_hbm, ow_hbm, idx_v, rows_v, sem):
    w = lax.axis_index("s") * _NC + lax.axis_index("c")
    base = w * _B_PER_W

    @pl.loop(0, _B_PER_W, step=_CH)
    def _(c):
        off = base + c
        pltpu.sync_copy(wid_hbm.at[pl.ds(off, _CH)], idx_v)
        pltpu.async_copy(word_hbm.at[idx_v], rows_v, sem).wait()
        pltpu.sync_copy(rows_v, ow_hbm.at[pl.ds(off, _CH)])


@functools.cache
def _gather_call():
    return functools.partial(
        pl.kernel,
        out_type=jax.ShapeDtypeStruct((_NH, HID), jnp.float32),
        mesh=plsc.VectorSubcoreMesh(core_axis_name="c", subcore_axis_name="s"),
        scratch_types=[
            pltpu.VMEM((_CH,), jnp.int32),
            pltpu.VMEM((_CH, HID), jnp.float32),
            pltpu.SemaphoreType.DMA,
        ],
    )(_gather_body)


def _make_ln_body(boff, aliased):
    def _ln_body(*refs):
        if aliased:
            (ss_ref, pid_ref, gw_ref, tab_ref, type_ref, g_ref, b_ref,
             _prev_ref, o_ref) = refs
        else:
            (ss_ref, pid_ref, gw_ref, tab_ref, type_ref, g_ref, b_ref,
             o_ref) = refs
        k = pl.program_id(0)
        b = pl.program_id(1) + boff
        ssv = ss_ref[b, k]
        pidc = pid_ref[...]

        slab = tab_ref[pl.ds(ssv * _SLAB, _KSLAB), :]
        lpid = pidc - ssv * _SLAB
        cols = lax.broadcasted_iota(jnp.int32, (_BLK, _KSLAB), 1)
        onehot = (lpid == cols).astype(jnp.bfloat16)
        acc = lax.dot_general(onehot, slab, (((1,), (0,)), ((), ())),
                              preferred_element_type=jnp.float32)

        padgate = jnp.where(ssv > 0, 1.0, 0.0)
        padrow = (padgate * (pidc == PAD).astype(jnp.float32)) \
            * tab_ref[PAD, :][None, :].astype(jnp.float32)
        x = gw_ref[...] + acc + padrow + type_ref[0, :][None, :]
        mean = jnp.mean(x, axis=-1, keepdims=True)
        xc = x - mean
        var = jnp.mean(xc * xc, axis=-1, keepdims=True)
        o_ref[...] = (xc * lax.rsqrt(var + EPS)) * g_ref[0, :][None, :] \
            + b_ref[0, :][None, :]
    return _ln_body


def _make_ln_call(boff, aliased):
    def _ob(k, b, ss):
        return (_NBLK * (b + boff) + k, 0)

    in_specs = [
        pl.BlockSpec((_BLK, 1), _ob),
        pl.BlockSpec((_BLK, HID), lambda k, b, ss: (_NBLK * b + k, 0)),
        pl.BlockSpec((_TAB_PAD, HID), lambda k, b, ss: (0, 0)),
        pl.BlockSpec((TYPEV, HID), lambda k, b, ss: (0, 0)),
        pl.BlockSpec((1, HID), lambda k, b, ss: (0, 0)),
        pl.BlockSpec((1, HID), lambda k, b, ss: (0, 0)),
    ]
    kwargs = {}
    if aliased:
        in_specs.append(pl.BlockSpec(memory_space=pl.ANY))
        kwargs["input_output_aliases"] = {7: 0}
    return pl.pallas_call(
        _make_ln_body(boff, aliased),
        grid_spec=pltpu.PrefetchScalarGridSpec(
            num_scalar_prefetch=1,
            grid=(_NBLK, B // 2),
            in_specs=in_specs,
            out_specs=pl.BlockSpec((_BLK, HID), _ob),
        ),
        out_shape=jax.ShapeDtypeStruct((N, HID), jnp.float32),
        **kwargs,
    )


_ln_call_a = _make_ln_call(0, aliased=False)
_ln_call_b = _make_ln_call(B // 2, aliased=True)


def kernel(input_ids, word_emb, pos_emb, type_emb, ln_gamma, ln_beta):
    position_ids, ss, tab = _posid_call(input_ids, pos_emb)
    ids_flat = input_ids.reshape(N)
    pid_col = position_ids.reshape(N, 1)
    gw_a = _gather_call()(word_emb, ids_flat[:_NH])
    gw_b = _gather_call()(word_emb, ids_flat[_NH:])
    g2 = ln_gamma.reshape(1, HID)
    b2 = ln_beta.reshape(1, HID)
    out_a = _ln_call_a(ss, pid_col, gw_a, tab, type_emb, g2, b2)
    out = _ln_call_b(ss, pid_col, gw_b, tab, type_emb, g2, b2, out_a)
    return out.reshape(B, S, HID), position_ids
